# Initial kernel scaffold; baseline (speedup 1.0000x reference)
import jax
import jax.numpy as jnp
from jax.experimental import pallas as pl


def kernel(x, edge_index, region_labels, W0, b0, u0, W1, b1, u1, W2, b2, u2, W3, b3, u3, W4, b4, u4):
    raise NotImplementedError("write your pallas kernel here")


# speedup vs baseline: 2154.0639x; 2154.0639x over previous
import functools

import jax
import jax.numpy as jnp
from jax import lax
from jax.experimental import pallas as pl
from jax.experimental.pallas import tpu as pltpu
from jax.experimental.pallas import tpu_sc as plsc

N = 10000
NP = 10112
E = 320000
NR = 64
NB = 64
SB = NP // NB
NT = 32
EPT = E // NT
EB = 256
FC = 512
BN = NP // 8

_f32 = jnp.float32
_i32 = jnp.int32


def _sc_mesh():
    return plsc.VectorSubcoreMesh(core_axis_name="c", subcore_axis_name="s")


def _wid():
    return lax.axis_index("s") * 2 + lax.axis_index("c")


def _compact_body(src_hbm, dst_hbm, lab_hbm,
                  csrc_hbm, cdst_hbm, cnt_hbm, degpart_hbm,
                  lab_v, sblk, dblk, csrc_v, cdst_v, cnt_v, deg_v):
    wid = _wid()
    BLK = 2000

    pltpu.sync_copy(lab_hbm, lab_v)

    def zfill(j, _):
        deg_v[pl.ds(j * 16, 16)] = jnp.zeros((16,), _f32)
        return 0
    lax.fori_loop(0, NP // 16, zfill, 0)

    base = wid * EPT
    cnt = jnp.int32(0)
    for blk in range(EPT // BLK):
        pltpu.sync_copy(src_hbm.at[pl.ds(base + blk * BLK, BLK)], sblk)
        pltpu.sync_copy(dst_hbm.at[pl.ds(base + blk * BLK, BLK)], dblk)

        def chunk(c, cnt):
            s16 = sblk[pl.ds(c * 16, 16)]
            d16 = dblk[pl.ds(c * 16, 16)]
            ls = plsc.load_gather(lab_v, [s16])
            ld = plsc.load_gather(lab_v, [d16])
            m = ls == ld
            plsc.store_compressed(csrc_v.at[pl.ds(cnt, 16)], s16, mask=m)
            plsc.store_compressed(cdst_v.at[pl.ds(cnt, 16)], d16, mask=m)
            return cnt + jnp.sum(m.astype(_i32))

        cnt = lax.fori_loop(0, BLK // 16, chunk, cnt)

    lane = lax.iota(_i32, 16)
    onehot0 = jnp.where(lane == 0, 1.0, 0.0).astype(_f32)

    def dgrp(g, _):
        d16 = jnp.clip(cdst_v[pl.ds(g * 16, 16)], 0, NP - 1)
        for e in range(16):
            dv = d16[e]
            inc = jnp.where(g * 16 + e < cnt, 1.0, 0.0).astype(_f32)
            deg_v[pl.ds(dv, 16)] = deg_v[pl.ds(dv, 16)] + inc * onehot0
        return 0
    lax.fori_loop(0, (cnt + 15) // 16, dgrp, 0)

    pltpu.sync_copy(csrc_v.at[pl.ds(0, EPT)], csrc_hbm.at[wid])
    pltpu.sync_copy(cdst_v.at[pl.ds(0, EPT)], cdst_hbm.at[wid])
    cnt_v[...] = jnp.broadcast_to(cnt, (16,)).astype(_i32)
    pltpu.sync_copy(cnt_v, cnt_hbm.at[wid])
    pltpu.sync_copy(deg_v.at[pl.ds(0, NP)], degpart_hbm.at[wid])


def _run_compact(src, dst, labels_pad):
    fn = pl.kernel(
        _compact_body,
        out_type=(jax.ShapeDtypeStruct((NT, EPT), _i32),
                  jax.ShapeDtypeStruct((NT, EPT), _i32),
                  jax.ShapeDtypeStruct((NT, 16), _i32),
                  jax.ShapeDtypeStruct((NT, NP), _f32)),
        mesh=_sc_mesh(),
        scratch_types=[
            pltpu.VMEM((NP,), _i32),
            pltpu.VMEM((2000,), _i32),
            pltpu.VMEM((2000,), _i32),
            pltpu.VMEM((EPT + 16,), _i32),
            pltpu.VMEM((EPT + 16,), _i32),
            pltpu.VMEM((16,), _i32),
            pltpu.VMEM((NP + 16,), _f32),
        ],
        compiler_params=pltpu.CompilerParams(use_tc_tiling_on_sc=False,
                                             needs_layout_passes=False),
    )
    return fn(src, dst, labels_pad)


def _msg_body(F, h_hbm, csrc_hbm, cdst_hbm, cnt_hbm, dinv_hbm, msg_hbm,
              dinv_v, cnts_v, esrc, edst, fsrc, fdst, fcoef, rows, msgb,
              semg):
    wid = _wid()
    lane = lax.iota(_i32, 16)

    pltpu.sync_copy(dinv_hbm, dinv_v)
    pltpu.sync_copy(cnt_hbm, cnts_v)

    def max_cnt(t, m):
        return jnp.maximum(m, cnts_v[t][0])
    maxn = lax.fori_loop(0, NT, max_cnt, jnp.int32(0))
    nblk = (maxn + EB - 1) // EB

    def proc_group(g, _):
        pltpu.async_copy(h_hbm.at[fsrc.at[pl.ds(g * 16, 16)]], rows,
                         semg).wait()
        dl16 = fdst[pl.ds(g * 16, 16)]
        cf16 = fcoef[pl.ds(g * 16, 16)]
        for e in range(16):
            dl = dl16[e]
            cf = cf16[e]
            for j in range(F // 16):
                sl = pl.ds(j * 16, 16)
                msgb[dl, sl] = msgb[dl, sl] + cf * rows[e, sl]
        return 0

    def bucket(bi, _):
        b = wid * 2 + bi
        lo = b * SB

        def zrow(r, _):
            for j in range(F // 16):
                msgb[r, pl.ds(j * 16, 16)] = jnp.zeros((16,), _f32)
            return 0
        lax.fori_loop(0, SB, zrow, 0)

        def flush(fcnt):
            lax.fori_loop(0, FC // 16, proc_group, 0)
            fsrc[pl.ds(0, 16)] = fsrc[pl.ds(FC, 16)]
            fdst[pl.ds(0, 16)] = fdst[pl.ds(FC, 16)]
            fcoef[pl.ds(0, 16)] = fcoef[pl.ds(FC, 16)]
            return fcnt - FC

        def blk_body(kk, fcnt):
            pltpu.sync_copy(csrc_hbm.at[:, pl.ds(kk * EB, EB)], esrc)
            pltpu.sync_copy(cdst_hbm.at[:, pl.ds(kk * EB, EB)], edst)

            def stripe(t, fcnt):
                nin = jnp.clip(cnts_v[t][0] - kk * EB, 0, EB)

                def chunk(c, fcnt):
                    s16r = esrc[t, pl.ds(c * 16, 16)]
                    d16r = edst[t, pl.ds(c * 16, 16)]
                    vmask = (c * 16 + lane) < nin
                    s16 = jnp.where(vmask, s16r, 0)
                    d16 = jnp.where(vmask, d16r, 0)
                    inb = (d16 >= lo) & (d16 < lo + SB)
                    m = vmask & inb
                    dsv = plsc.load_gather(dinv_v, [s16])
                    ddv = plsc.load_gather(dinv_v, [d16])
                    cf16 = dsv * ddv
                    plsc.store_compressed(fsrc.at[pl.ds(fcnt, 16)], s16, mask=m)
                    plsc.store_compressed(fdst.at[pl.ds(fcnt, 16)],
                                          d16 - lo, mask=m)
                    plsc.store_compressed(fcoef.at[pl.ds(fcnt, 16)], cf16, mask=m)
                    fcnt2 = fcnt + jnp.sum(m.astype(_i32))
                    return lax.cond(fcnt2 >= FC, flush, lambda f: f, fcnt2)

                nch = (nin + 15) // 16
                return lax.fori_loop(0, nch, chunk, fcnt)

            return lax.fori_loop(0, NT, stripe, fcnt)

        fcnt = lax.fori_loop(0, nblk, blk_body, jnp.int32(0))

        rem = fcnt % 16
        toff = fcnt - rem
        okm = lane < rem
        fsrc[pl.ds(toff, 16)] = jnp.where(okm, fsrc[pl.ds(toff, 16)], 0)
        fdst[pl.ds(toff, 16)] = jnp.where(okm, fdst[pl.ds(toff, 16)], 0)
        fcoef[pl.ds(toff, 16)] = jnp.where(okm, fcoef[pl.ds(toff, 16)], 0.0)
        lax.fori_loop(0, (fcnt + 15) // 16, proc_group, 0)

        pltpu.sync_copy(msgb, msg_hbm.at[pl.ds(lo, SB)])
        return 0

    lax.fori_loop(0, 2, bucket, 0)


def _run_msg(h, csrc, cdst, cnts, dinv, F):
    fn = pl.kernel(
        functools.partial(_msg_body, F),
        out_type=jax.ShapeDtypeStruct((NP, F), _f32),
        mesh=_sc_mesh(),
        scratch_types=[
            pltpu.VMEM((NP,), _f32),
            pltpu.VMEM((NT, 16), _i32),
            pltpu.VMEM((NT, EB), _i32),
            pltpu.VMEM((NT, EB), _i32),
            pltpu.VMEM((FC + 16,), _i32),
            pltpu.VMEM((FC + 16,), _i32),
            pltpu.VMEM((FC + 16,), _f32),
            pltpu.VMEM((16, F), _f32),
            pltpu.VMEM((SB, F), _f32),
            pltpu.SemaphoreType.DMA,
        ],
        compiler_params=pltpu.CompilerParams(use_tc_tiling_on_sc=False,
                                             needs_layout_passes=False),
    )
    return fn(h, csrc, cdst, cnts, dinv)


def _sn_matmul(a, W, u):
    wtu = jnp.dot(u, W, preferred_element_type=_f32)
    v = wtu / (jnp.sqrt(jnp.sum(wtu * wtu)) + 1e-12)
    wv = jax.lax.dot_general(W, v, (((1,), (1,)), ((), ())),
                             preferred_element_type=_f32)
    nv = jnp.sqrt(jnp.sum(wv * wv))
    sigma = nv * nv / (nv + 1e-12)
    h = jax.lax.dot_general(a, W, (((1,), (1,)), ((), ())),
                            preferred_element_type=_f32)
    return h / sigma


def _first_body(x_ref, W_ref, u_ref, out_ref):
    out_ref[...] = _sn_matmul(x_ref[...], W_ref[...], u_ref[...])


def _run_first(x_pad, W0, u0):
    Fout, Fin = W0.shape
    return pl.pallas_call(
        _first_body,
        grid=(8,),
        in_specs=[
            pl.BlockSpec((BN, Fin), lambda i: (i, 0)),
            pl.BlockSpec((Fout, Fin), lambda i: (0, 0)),
            pl.BlockSpec((1, Fout), lambda i: (0, 0)),
        ],
        out_specs=pl.BlockSpec((BN, Fout), lambda i: (i, 0)),
        out_shape=jax.ShapeDtypeStruct((NP, Fout), _f32),
    )(x_pad, W0, u0)


def _mid_body(msg_ref, h_ref, d2_ref, b_ref, W_ref, u_ref, out_ref):
    o = msg_ref[...] + d2_ref[...] * h_ref[...] + b_ref[...]
    mu = jnp.mean(o, axis=1, keepdims=True)
    var = jnp.mean((o - mu) * (o - mu), axis=1, keepdims=True)
    a = (o - mu) * lax.rsqrt(var + 1e-5)
    a = jnp.where(a >= 0, a, 0.2 * a)
    out_ref[...] = _sn_matmul(a, W_ref[...], u_ref[...])


def _run_mid(msg, h, dinv2c, b, W, u):
    Fout, Fin = W.shape
    return pl.pallas_call(
        _mid_body,
        grid=(8,),
        in_specs=[
            pl.BlockSpec((BN, Fin), lambda i: (i, 0)),
            pl.BlockSpec((BN, Fin), lambda i: (i, 0)),
            pl.BlockSpec((BN, 1), lambda i: (i, 0)),
            pl.BlockSpec((1, Fin), lambda i: (0, 0)),
            pl.BlockSpec((Fout, Fin), lambda i: (0, 0)),
            pl.BlockSpec((1, Fout), lambda i: (0, 0)),
        ],
        out_specs=pl.BlockSpec((BN, Fout), lambda i: (i, 0)),
        out_shape=jax.ShapeDtypeStruct((NP, Fout), _f32),
    )(msg, h, dinv2c, b, W, u)


def _dinv_body(parts_ref, dinv_ref, dinv2_ref):
    deg = jnp.sum(parts_ref[...], axis=0) + 1.0
    di = lax.rsqrt(deg)
    dinv_ref[...] = di
    dinv2_ref[...] = di * di


def _run_dinv(degpart):
    parts = degpart.reshape(NT, 79, 128)
    return pl.pallas_call(
        _dinv_body,
        out_shape=(jax.ShapeDtypeStruct((79, 128), _f32),
                   jax.ShapeDtypeStruct((79, 128), _f32)),
    )(parts)


def _score_body(msg_ref, h_ref, d2_ref, b_ref, lab_ref, out_ref):
    o = msg_ref[...] + d2_ref[...] * h_ref[...] + b_ref[...]
    labs = lab_ref[...]
    scores = []
    for r in range(NR):
        mk = (labs == r).astype(_f32)
        scores.append(jnp.sum(mk * o) / jnp.sum(mk))
    out_ref[...] = jnp.stack(scores)[None, :]


def _run_score(msgc, hc, d2c, b4, labc):
    return pl.pallas_call(
        _score_body,
        out_shape=jax.ShapeDtypeStruct((1, NR), _f32),
    )(msgc, hc, d2c, b4, labc)


def kernel(x, edge_index, region_labels,
           W0, b0, u0, W1, b1, u1, W2, b2, u2, W3, b3, u3, W4, b4, u4):
    f32 = _f32
    x_pad = jnp.zeros((NP, 128), f32).at[:N].set(x.astype(f32))
    labels_pad = jnp.full((NP,), NR, _i32).at[:N].set(
        region_labels.astype(_i32))
    src = edge_index[0].astype(_i32)
    dst = edge_index[1].astype(_i32)

    W4p = jnp.zeros((16, 512), f32).at[0].set(W4[0].astype(f32))
    u4p = jnp.zeros((16,), f32).at[0].set(u4[0].astype(f32))
    b4p = jnp.zeros((16,), f32).at[0].set(b4[0].astype(f32))

    Ws = [W0, W1, W2, W3, W4p]
    bs = [b0, b1, b2, b3, b4p]
    us = [u0, u1, u2, u3, u4p]

    csrc, cdst, cnts, degpart = _run_compact(src, dst, labels_pad)
    dinv2d, dinv2_2d = _run_dinv(degpart)
    dinv = dinv2d.reshape(NP)
    dinv2c = dinv2_2d.reshape(NP, 1)

    h = _run_first(x_pad, Ws[0], us[0].reshape(1, -1))
    for i in range(5):
        F = h.shape[1]
        msg = _run_msg(h, csrc, cdst, cnts, dinv, F)
        if i < 4:
            h = _run_mid(msg, h, dinv2c, bs[i].reshape(1, -1),
                         Ws[i + 1], us[i + 1].reshape(1, -1))
        else:
            out2d = _run_score(msg[:, 0].reshape(79, 128),
                               h[:, 0].reshape(79, 128),
                               dinv2_2d, b4p[0:1].reshape(1, 1),
                               labels_pad.reshape(79, 128))
    return out2d[0]
